# baseline (speedup 1.0000x reference)
import functools

import jax
import jax.numpy as jnp
from jax.experimental import pallas as pl


N = 10000
E = 320000
D = 128


def _uv_body(x_ref, wd_ref, wb_ref, b1_ref, u_ref, v_ref):
    x = x_ref[...]
    u_ref[...] = jnp.dot(x, wd_ref[...], preferred_element_type=jnp.float32) + b1_ref[...]
    v_ref[...] = jnp.dot(x, wb_ref[...], preferred_element_type=jnp.float32)


@jax.jit
def _uv(x, W1, b1):
    wa = W1[:D]
    wb = W1[D:]
    wd = wa - wb
    nblk = 10
    bs = N // nblk
    return pl.pallas_call(
        _uv_body,
        grid=(nblk,),
        in_specs=[
            pl.BlockSpec((bs, D), lambda i: (i, 0)),
            pl.BlockSpec((D, D), lambda i: (0, 0)),
            pl.BlockSpec((D, D), lambda i: (0, 0)),
            pl.BlockSpec((D,), lambda i: (0,)),
        ],
        out_specs=[
            pl.BlockSpec((bs, D), lambda i: (i, 0)),
            pl.BlockSpec((bs, D), lambda i: (i, 0)),
        ],
        out_shape=[
            jax.ShapeDtypeStruct((N, D), jnp.float32),
            jax.ShapeDtypeStruct((N, D), jnp.float32),
        ],
    )(x, wd, wb, b1)


def _h_body(g_ref, w2_ref, b2_ref, h_ref):
    h_ref[...] = (
        jnp.dot(jnp.maximum(g_ref[...], 0.0), w2_ref[...],
                preferred_element_type=jnp.float32)
        + b2_ref[...]
    )


@jax.jit
def _edge_mlp2(g, W2, b2):
    nblk = 625
    bs = E // nblk
    return pl.pallas_call(
        _h_body,
        grid=(nblk,),
        in_specs=[
            pl.BlockSpec((bs, D), lambda i: (i, 0)),
            pl.BlockSpec((D, D), lambda i: (0, 0)),
            pl.BlockSpec((D,), lambda i: (0,)),
        ],
        out_specs=pl.BlockSpec((bs, D), lambda i: (i, 0)),
        out_shape=jax.ShapeDtypeStruct((E, D), jnp.float32),
    )(g, W2, b2)


def _layer(x, src, dst, W1, b1, W2, b2, final):
    u, v = _uv(x, W1, b1)
    g = u[dst] + v[src]
    h = _edge_mlp2(g, W2, b2)
    agg = jax.ops.segment_max(h, dst, num_segments=N)
    if final:
        return jnp.where(jnp.isneginf(agg), 0.0, agg)
    return jnp.maximum(agg, 0.0)


def kernel(x, edge_index, W1_in, b1_in, W2_in, b2_in, W1_hid, b1_hid,
           W2_hid, b2_hid, W1_out, b1_out, W2_out, b2_out):
    src = edge_index[0]
    dst = edge_index[1]
    x = _layer(x, src, dst, W1_in, b1_in, W2_in, b2_in, final=False)
    x = _layer(x, src, dst, W1_hid, b1_hid, W2_hid, b2_hid, final=False)
    x = _layer(x, src, dst, W1_out, b1_out, W2_out, b2_out, final=True)
    return x
